# baseline (speedup 1.0000x reference)
import functools

import jax
import jax.numpy as jnp
from jax import lax
from jax.experimental import pallas as pl
from jax.experimental.pallas import tpu as pltpu
from jax.experimental.pallas import tpu_sc as plsc

N = 30000
D = 128
H = 128
CW = 128
OUT = 64
E = 300000

NC = 2
NS = 16
NQ = 4
QS = N // NQ
RTQ = 8000
NP = NQ * RTQ
TRASH = QS
CP = 512
CPL = RTQ - (NS - 1) * CP

CH = 128
NCH = 160
NCH2 = NCH // 2
EPT = NCH * CH
EPAD = EPT * NS

BS = 1000
GRID = NP // BS


@functools.cache
def _mesh():
    return plsc.VectorSubcoreMesh(
        core_axis_name="c", subcore_axis_name="s",
        num_cores=NC, num_subcores=NS)


def _zero_own(zeros_hbm, acc, s):
    @pl.when(s < NS - 1)
    def _():
        pltpu.sync_copy(zeros_hbm, acc.at[pl.ds(s * CP, CP)])

    @pl.when(s == NS - 1)
    def _():
        pltpu.sync_copy(zeros_hbm.at[pl.ds(0, CPL)],
                        acc.at[pl.ds((NS - 1) * CP, CPL)])


def _copy_out(acc, out_hbm, s, base):
    @pl.when(s < NS - 1)
    def _():
        pltpu.sync_copy(acc.at[pl.ds(s * CP, CP)],
                        out_hbm.at[pl.ds(base + s * CP, CP)])

    @pl.when(s == NS - 1)
    def _():
        pltpu.sync_copy(acc.at[pl.ds((NS - 1) * CP, CPL)],
                        out_hbm.at[pl.ds(base + (NS - 1) * CP, CPL)])


def _spmm_body(x_hbm, src_hbm, dst_hbm, zeros_hbm, out_hbm,
               src_v, dst_v, didx, buf0, buf1, acc, sem0, sem1):
    c = lax.axis_index("c")
    s = lax.axis_index("s")

    bufs = (buf0, buf1)
    sems = (sem0, sem1)

    def issue(j, b):
        pltpu.async_copy(x_hbm.at[src_v.at[j]], bufs[b], sems[b])

    def wait(j, b):
        pltpu.make_async_copy(x_hbm.at[src_v.at[j]], bufs[b], sems[b]).wait()

    def scat(j, b):
        for k in range(CH // 16):
            didx[pl.ds(k * 16, 16)] = dst_v[j, pl.ds(k * 16, 16)]
        pltpu.sync_copy(bufs[b], acc.at[didx], add=True)

    def chunk_loop(i, carry):
        j = 2 * i
        issue(j + 1, 1)
        wait(j, 0)
        scat(j, 0)

        @pl.when(j + 2 < NCH2)
        def _():
            issue(j + 2, 0)

        wait(j + 1, 1)
        scat(j + 1, 1)
        return carry

    for q in range(2):
        qg = c * 2 + q
        _zero_own(zeros_hbm, acc, s)
        plsc.subcore_barrier()
        for hb in range(2):
            pltpu.sync_copy(
                src_hbm.at[pl.ds(s * NCH + hb * NCH2, NCH2)], src_v)
            pltpu.sync_copy(
                dst_hbm.at[pl.ds((qg * NS + s) * NCH + hb * NCH2, NCH2)],
                dst_v)
            issue(0, 0)
            lax.fori_loop(0, NCH2 // 2, chunk_loop, 0)
        plsc.subcore_barrier()
        _copy_out(acc, out_hbm, s, qg * RTQ)


@functools.cache
def _spmm_kernel():
    return pl.kernel(
        _spmm_body,
        out_type=jax.ShapeDtypeStruct((NP, H), jnp.float32),
        mesh=_mesh(),
        scratch_types=[
            pltpu.VMEM((NCH2, CH), jnp.int32),
            pltpu.VMEM((NCH2, CH), jnp.int32),
            pltpu.VMEM((CH,), jnp.int32),
            pltpu.VMEM((CH, H), jnp.float32),
            pltpu.VMEM((CH, H), jnp.float32),
            pltpu.VMEM_SHARED((RTQ, H), jnp.float32),
            pltpu.SemaphoreType.DMA,
            pltpu.SemaphoreType.DMA,
        ],
    )


def _spmm(x, src_r, dst_r, zeros_h):
    return _spmm_kernel()(x, src_r, dst_r, zeros_h)


def _cnt_body(dst_hbm, ones_hbm, zeros_hbm, out_hbm, dst_v, didx, ones_v, acc):
    c = lax.axis_index("c")
    s = lax.axis_index("s")
    pltpu.sync_copy(ones_hbm, ones_v)

    def body(i, carry):
        for k in range(CH // 16):
            didx[pl.ds(k * 16, 16)] = dst_v[i, pl.ds(k * 16, 16)]
        pltpu.sync_copy(ones_v, acc.at[didx], add=True)
        return carry

    for q in range(2):
        qg = c * 2 + q
        _zero_own(zeros_hbm, acc, s)
        plsc.subcore_barrier()
        for hb in range(2):
            pltpu.sync_copy(
                dst_hbm.at[pl.ds((qg * NS + s) * NCH + hb * NCH2, NCH2)],
                dst_v)
            lax.fori_loop(0, NCH2, body, 0)
        plsc.subcore_barrier()
        _copy_out(acc, out_hbm, s, qg * RTQ)


@functools.cache
def _cnt_kernel():
    return pl.kernel(
        _cnt_body,
        out_type=jax.ShapeDtypeStruct((NP, CW), jnp.float32),
        mesh=_mesh(),
        scratch_types=[
            pltpu.VMEM((NCH2, CH), jnp.int32),
            pltpu.VMEM((CH,), jnp.int32),
            pltpu.VMEM((CH, CW), jnp.float32),
            pltpu.VMEM_SHARED((RTQ, CW), jnp.float32),
        ],
    )


def _cnt(dst_r, ones_c, zeros_c):
    return _cnt_kernel()(dst_r, ones_c, zeros_c)


def _proj_body(x_ref, w_ref, b_ref, o_ref):
    y = jnp.dot(x_ref[...], w_ref[...], preferred_element_type=jnp.float32)
    o_ref[...] = jnp.maximum(y + b_ref[...], 0.0)


def _proj(x, w, b):
    return pl.pallas_call(
        _proj_body,
        grid=(GRID,),
        in_specs=[
            pl.BlockSpec((BS, D), lambda r: (r, 0)),
            pl.BlockSpec((D, H), lambda r: (0, 0)),
            pl.BlockSpec((1, H), lambda r: (0, 0)),
        ],
        out_specs=pl.BlockSpec((BS, H), lambda r: (r, 0)),
        out_shape=jax.ShapeDtypeStruct((NP, H), jnp.float32),
    )(x, w, b.reshape(1, H))


def _comb_body(acc_ref, cnt_ref, x_ref, wl_ref, bl_ref, wr_ref, o_ref):
    cnt = cnt_ref[:, :1]
    mean = acc_ref[...] / jnp.maximum(cnt, 1.0)
    y = (jnp.dot(mean, wl_ref[...], preferred_element_type=jnp.float32)
         + bl_ref[...]
         + jnp.dot(x_ref[...], wr_ref[...], preferred_element_type=jnp.float32))
    o_ref[...] = jnp.maximum(y, 0.0)


def _combine(acc, cnt, x, wl, bl, wr):
    return pl.pallas_call(
        _comb_body,
        grid=(GRID,),
        in_specs=[
            pl.BlockSpec((BS, H), lambda r: (r, 0)),
            pl.BlockSpec((BS, CW), lambda r: (r, 0)),
            pl.BlockSpec((BS, H), lambda r: (r, 0)),
            pl.BlockSpec((H, H), lambda r: (0, 0)),
            pl.BlockSpec((1, H), lambda r: (0, 0)),
            pl.BlockSpec((H, H), lambda r: (0, 0)),
        ],
        out_specs=pl.BlockSpec((BS, H), lambda r: (r, 0)),
        out_shape=jax.ShapeDtypeStruct((NP, H), jnp.float32),
    )(acc, cnt, x, wl, bl.reshape(1, H), wr)


def _final_body(x_ref, w_ref, b_ref, o_ref):
    o_ref[...] = (jnp.dot(x_ref[...], w_ref[...],
                          preferred_element_type=jnp.float32) + b_ref[...])


def _final(x, w, b):
    return pl.pallas_call(
        _final_body,
        grid=(GRID,),
        in_specs=[
            pl.BlockSpec((BS, H), lambda r: (r, 0)),
            pl.BlockSpec((H, OUT), lambda r: (0, 0)),
            pl.BlockSpec((1, OUT), lambda r: (0, 0)),
        ],
        out_specs=pl.BlockSpec((BS, OUT), lambda r: (r, 0)),
        out_shape=jax.ShapeDtypeStruct((NP, OUT), jnp.float32),
    )(x, w, b.reshape(1, OUT))


def _pad_rows(x):
    x4 = x.reshape(NQ, QS, x.shape[1])
    return jnp.pad(x4, ((0, 0), (0, RTQ - QS), (0, 0))).reshape(NP, -1)


def _unpad_rows(y):
    return y.reshape(NQ, RTQ, y.shape[1])[:, :QS].reshape(N, -1)


def _prep_edges(edge_index):
    src = edge_index[0].astype(jnp.int32)
    dst = edge_index[1].astype(jnp.int32)
    src = jnp.concatenate([src, jnp.zeros((EPAD - E,), jnp.int32)])
    dst = jnp.concatenate([dst, jnp.full((EPAD - E,), N, jnp.int32)])
    src = src + (RTQ - QS) * (src // QS)
    locs = []
    for q in range(NQ):
        lo, hi = q * QS, (q + 1) * QS
        inside = (dst >= lo) & (dst < hi)
        locs.append(jnp.where(inside, dst - lo, TRASH).reshape(NS * NCH, CH))
    return src.reshape(NS * NCH, CH), jnp.concatenate(locs, axis=0)


def kernel(x_user, x_item, edge_index_user_item, edge_index_item_user,
           lin_user_w, lin_user_b, lin_item_w, lin_item_b,
           Wl, bl, Wr, final_w, final_b):
    src_ui, dst_ui = _prep_edges(edge_index_user_item)
    src_iu, dst_iu = _prep_edges(edge_index_item_user)

    zeros_h = jnp.zeros((CP, H), jnp.float32)
    zeros_c = jnp.zeros((CP, CW), jnp.float32)
    ones_c = jnp.ones((CH, CW), jnp.float32)

    cnt_i = _cnt(dst_ui, ones_c, zeros_c)
    cnt_u = _cnt(dst_iu, ones_c, zeros_c)

    xu = _proj(_pad_rows(x_user), lin_user_w, lin_user_b)
    xi = _proj(_pad_rows(x_item), lin_item_w, lin_item_b)

    for l in range(3):
        acc_i = _spmm(xu, src_ui, dst_ui, zeros_h)
        acc_u = _spmm(xi, src_iu, dst_iu, zeros_h)
        xi_new = _combine(acc_i, cnt_i, xi, Wl[l, 0], bl[l, 0], Wr[l, 0])
        xu_new = _combine(acc_u, cnt_u, xu, Wl[l, 1], bl[l, 1], Wr[l, 1])
        xu, xi = xu_new, xi_new

    out_user = _unpad_rows(_final(xu, final_w, final_b))
    out_item = _unpad_rows(_final(xi, final_w, final_b))
    return (out_user, out_item)
